# baseline (speedup 1.0000x reference)
import functools

import jax
import jax.numpy as jnp
from jax import lax
from jax.experimental import pallas as pl
from jax.experimental.pallas import tpu as pltpu
from jax.experimental.pallas import tpu_sc as plsc

NC = 2
NS = 16
NW = NC * NS

_B = 64

_RT = 640
_SZ = 128
_KD = 1000
_KA = 400
_NIB = 4
_NRB = 3


def _sc_mesh():
    return plsc.VectorSubcoreMesh(
        core_axis_name="c", subcore_axis_name="s", num_cores=NC,
        num_subcores=NS)


def _vrsqrt(x):
    i = lax.bitcast_convert_type(x, jnp.int32)
    i = 0x5F3759DF - lax.shift_right_logical(i, 1)
    y = lax.bitcast_convert_type(i, jnp.float32)
    for _ in range(3):
        y = y * (1.5 - 0.5 * x * y * y)
    return y


def _edge_pipeline(ei_hbm, table, acc_sh, src_v, dst_v, rows_v,
                   isems, gsems, ssems, base, nchunk, K):
    def idx_start(i):
        off = pl.multiple_of(base + i * K, K)
        j = i % _NIB
        return (
            pltpu.async_copy(ei_hbm.at[0, pl.ds(off, K)], src_v.at[j],
                             isems[j]),
            pltpu.async_copy(ei_hbm.at[1, pl.ds(off, K)], dst_v.at[j],
                             isems[j]),
        )

    def gather_start(i):
        return pltpu.async_copy(table.at[src_v.at[i % _NIB]],
                                rows_v.at[i % _NRB], gsems[i % _NRB])

    def scat_start(i):
        return pltpu.async_copy(rows_v.at[i % _NRB],
                                acc_sh.at[dst_v.at[i % _NIB]],
                                ssems[i % _NRB], add=True)

    idx_d = {i: idx_start(i) for i in range(min(3, nchunk))}
    plsc.subcore_barrier()
    gat_d = {}
    for i in range(min(2, nchunk)):
        for d in idx_d[i]:
            d.wait()
        gat_d[i] = gather_start(i)

    scat_d = {}
    for i in range(nchunk):
        gat_d[i].wait()
        if i >= 1:
            scat_d[i - 1].wait()
        scat_d[i] = scat_start(i)
        if i + 2 < nchunk:
            for d in idx_d[i + 2]:
                d.wait()
            gat_d[i + 2] = gather_start(i + 2)
        if i + 3 < nchunk:
            idx_d[i + 3] = idx_start(i + 3)
    scat_d[nchunk - 1].wait()


def _make_layer1_kernel(N, E, H):
    ept_deg = E // NS
    nch_deg = ept_deg // _KD
    ept = E // NW
    nch = ept // _KA
    nsub = _RT // _SZ

    @functools.partial(
        pl.kernel,
        out_type=(
            jax.ShapeDtypeStruct((NC, N, H), jnp.float32),
            jax.ShapeDtypeStruct((N,), jnp.float32),
            jax.ShapeDtypeStruct((NC, N, H), jnp.float32),
        ),
        mesh=_sc_mesh(),
        scratch_types=[
            pltpu.VMEM((_NIB, _KD), jnp.int32),
            pltpu.VMEM((_KD,), jnp.float32),
            pltpu.VMEM((_NIB, _KA), jnp.int32),
            pltpu.VMEM((_NIB, _KA), jnp.int32),
            pltpu.VMEM((_NRB, _KA, H), jnp.float32),
            pltpu.VMEM((_RT,), jnp.float32),
            pltpu.VMEM((_RT,), jnp.float32),
            pltpu.VMEM_SHARED((N,), jnp.float32),
            pltpu.VMEM_SHARED((N, H), jnp.float32),
        ]
        + [pltpu.SemaphoreType.DMA] * (_NIB + 2 * _NRB + 2),
        compiler_params=pltpu.CompilerParams(use_tc_tiling_on_sc=False),
    )
    def layer1(h1_hbm, ei_hbm, ones_hbm, zeros1_hbm, zerosh_hbm,
               p_hbm, dinv_hbm, tbl_hbm,
               didx_v, ones_v, src_v, dst_v, rows_v, deg_v, dinv_v,
               deg_sh, acc_sh, *sems):
        isems = sems[:_NIB]
        gsems = sems[_NIB:_NIB + _NRB]
        ssems = sems[_NIB + _NRB:_NIB + 2 * _NRB]
        zsem = sems[_NIB + 2 * _NRB]
        z2sem = sems[_NIB + 2 * _NRB + 1]
        c = lax.axis_index("c")
        s = lax.axis_index("s")
        start = pl.multiple_of(jnp.minimum(s * _RT, N - _RT), 80)

        dbase = s * ept_deg

        def didx_start(i):
            return pltpu.async_copy(
                ei_hbm.at[1, pl.ds(pl.multiple_of(dbase + i * _KD, _KD),
                                   _KD)],
                didx_v.at[i % _NIB], isems[i % _NIB])

        @pl.when(s == 0)
        def _():
            pltpu.async_copy(zeros1_hbm, deg_sh, zsem)

        didx_d = {i: didx_start(i) for i in range(min(3, nch_deg))}
        pltpu.sync_copy(ones_hbm, ones_v)
        zd = pltpu.async_copy(zerosh_hbm.at[pl.ds(start, _RT)],
                              acc_sh.at[pl.ds(start, _RT)], z2sem)

        @pl.when(s == 0)
        def _():
            pltpu.make_async_copy(zeros1_hbm, deg_sh, zsem).wait()
        plsc.subcore_barrier()

        dscat_d = {}
        for i in range(nch_deg):
            didx_d[i].wait()
            if i >= 1:
                dscat_d[i - 1].wait()
            dscat_d[i] = pltpu.async_copy(
                ones_v, deg_sh.at[didx_v.at[i % _NIB]], ssems[i % 2],
                add=True)
            if i + 3 < nch_deg:
                didx_d[i + 3] = didx_start(i + 3)
        dscat_d[nch_deg - 1].wait()
        plsc.subcore_barrier()

        pltpu.sync_copy(deg_sh.at[pl.ds(start, _RT)], deg_v)
        for i in range(_RT // 16):
            sl = pl.ds(i * 16, 16)
            dinv_v[sl] = _vrsqrt(deg_v[sl] + 1.0)

        @pl.when(c == 0)
        def _():
            pltpu.sync_copy(dinv_v, dinv_hbm.at[pl.ds(start, _RT)])

        def ld_start(p):
            r0 = start + p * _SZ
            return pltpu.async_copy(
                h1_hbm.at[pl.ds(r0, _SZ)],
                rows_v.at[p % _NRB, pl.ds(0, _SZ)], gsems[p % _NRB])

        def st_start(p):
            r0 = start + p * _SZ
            return pltpu.async_copy(
                rows_v.at[p % _NRB, pl.ds(0, _SZ)],
                tbl_hbm.at[c, pl.ds(r0, _SZ)], ssems[p % _NRB])

        ld_d = {p: ld_start(p) for p in range(min(2, nsub))}
        st_d = {}
        for p in range(nsub):
            ld_d[p].wait()
            if p >= 1:
                st_d[p - 1].wait()
            if p + 2 < nsub:
                ld_d[p + 2] = ld_start(p + 2)

            def scale_grp(i, carry):
                dvec = dinv_v[pl.ds(
                    pl.multiple_of(p * _SZ + i * 16, 16), 16)]
                for j in range(16):
                    r = i * 16 + j
                    dv = dvec[j]
                    for cc in range(H // 16):
                        sl = pl.ds(cc * 16, 16)
                        rows_v[p % _NRB, r, sl] = rows_v[p % _NRB, r, sl] * dv
                return carry
            lax.fori_loop(0, _SZ // 16, scale_grp, 0)
            st_d[p] = st_start(p)
        st_d[nsub - 1].wait()
        zd.wait()

        base = (c * NS + s) * ept
        _edge_pipeline(ei_hbm, tbl_hbm.at[c], acc_sh, src_v, dst_v, rows_v,
                       isems, gsems, ssems, base, nch, _KA)

        plsc.subcore_barrier()
        pltpu.sync_copy(acc_sh.at[pl.ds(start, _RT)],
                        p_hbm.at[c, pl.ds(start, _RT)])

    return layer1


def _make_layer2_kernel(N, E, H):
    ept = E // NW
    nch = ept // _KA
    nsub = _RT // _SZ

    @functools.partial(
        pl.kernel,
        out_type=(
            jax.ShapeDtypeStruct((NC, N, H), jnp.float32),
            jax.ShapeDtypeStruct((N, H), jnp.float32),
            jax.ShapeDtypeStruct((NC, N, H), jnp.float32),
        ),
        mesh=_sc_mesh(),
        scratch_types=[
            pltpu.VMEM((_NIB, _KA), jnp.int32),
            pltpu.VMEM((_NIB, _KA), jnp.int32),
            pltpu.VMEM((_NRB, _KA, H), jnp.float32),
            pltpu.VMEM((_RT,), jnp.float32),
            pltpu.VMEM((H,), jnp.float32),
            pltpu.VMEM_SHARED((N, H), jnp.float32),
        ]
        + [pltpu.SemaphoreType.DMA] * (_NIB + 2 * _NRB + 1),
        compiler_params=pltpu.CompilerParams(use_tc_tiling_on_sc=False),
    )
    def layer2(p1_hbm, h1_hbm, dinv_hbm, b1_hbm, ei_hbm, zerosh_hbm,
               p_hbm, relu1_hbm, tbl_hbm,
               src_v, dst_v, rows_v, dinv_v, b1_v, acc_sh, *sems):
        isems = sems[:_NIB]
        gsems = sems[_NIB:_NIB + _NRB]
        ssems = sems[_NIB + _NRB:_NIB + 2 * _NRB]
        zsem = sems[_NIB + 2 * _NRB]
        c = lax.axis_index("c")
        s = lax.axis_index("s")
        start = pl.multiple_of(jnp.minimum(s * _RT, N - _RT), 80)

        zd = pltpu.async_copy(zerosh_hbm.at[pl.ds(start, _RT)],
                              acc_sh.at[pl.ds(start, _RT)], zsem)
        pltpu.sync_copy(dinv_hbm.at[pl.ds(start, _RT)], dinv_v)
        pltpu.sync_copy(b1_hbm, b1_v)

        ldsems = list(isems) + [gsems[0], gsems[1]]

        def ld_start(p):
            m = p % 2
            r0 = start + p * _SZ
            reg = pl.ds(m * _SZ, _SZ)
            return (
                pltpu.async_copy(p1_hbm.at[0, pl.ds(r0, _SZ)],
                                 rows_v.at[0, reg], ldsems[3 * m]),
                pltpu.async_copy(p1_hbm.at[1, pl.ds(r0, _SZ)],
                                 rows_v.at[1, reg], ldsems[3 * m + 1]),
                pltpu.async_copy(h1_hbm.at[pl.ds(r0, _SZ)],
                                 rows_v.at[2, reg], ldsems[3 * m + 2]),
            )

        def st_start(p):
            m = p % 2
            r0 = start + p * _SZ
            reg = pl.ds(m * _SZ, _SZ)
            return pltpu.async_copy(rows_v.at[1, reg],
                                    tbl_hbm.at[c, pl.ds(r0, _SZ)],
                                    ssems[m])

        ld_d = {p: ld_start(p) for p in range(min(2, nsub))}
        st_d = {}
        for p in range(nsub):
            m = p % 2
            for d in ld_d[p]:
                d.wait()

            def mid_grp(i, carry):
                dvec = dinv_v[pl.ds(
                    pl.multiple_of(p * _SZ + i * 16, 16), 16)]
                for j in range(16):
                    r = m * _SZ + i * 16 + j
                    dv = dvec[j]
                    for cc in range(H // 16):
                        sl = pl.ds(cc * 16, 16)
                        t = (rows_v[0, r, sl] + rows_v[1, r, sl]) * dv \
                            + rows_v[2, r, sl] * (dv * dv) + b1_v[sl]
                        t = jnp.maximum(t, 0.0)
                        rows_v[0, r, sl] = t
                        rows_v[1, r, sl] = t * dv
                return carry
            lax.fori_loop(0, _SZ // 16, mid_grp, 0)

            @pl.when(c == 0)
            def _():
                pltpu.sync_copy(rows_v.at[0, pl.ds(m * _SZ, _SZ)],
                                relu1_hbm.at[pl.ds(start + p * _SZ, _SZ)])
            st_d[p] = st_start(p)
            if p + 2 < nsub:
                st_d[p].wait()
                ld_d[p + 2] = ld_start(p + 2)
        for p in (nsub - 2, nsub - 1):
            if p >= 0 and p + 2 >= nsub:
                st_d[p].wait()
        zd.wait()

        base = (c * NS + s) * ept
        _edge_pipeline(ei_hbm, tbl_hbm.at[c], acc_sh, src_v, dst_v, rows_v,
                       isems, gsems, ssems, base, nch, _KA)

        plsc.subcore_barrier()
        pltpu.sync_copy(acc_sh.at[pl.ds(start, _RT)],
                        p_hbm.at[c, pl.ds(start, _RT)])

    return layer2


def _pre_body(x_ref, w1_ref, h_ref):
    h_ref[...] = jnp.dot(x_ref[...], w1_ref[...],
                         preferred_element_type=jnp.float32)


def _make_post_body(nblocks, H):
    def post_body(p_ref, relu1_ref, dinv_ref, w2_ref, b2_ref,
                  batch_col_ref, batch_row_ref, wf1_ref, bf1_ref,
                  wf2_ref, bf2_ref, out_ref, xmax_acc, sum_acc, cnt_acc):
        k = pl.program_id(0)
        bs = relu1_ref.shape[0]

        dinv = dinv_ref[...]
        agg = (p_ref[0] + p_ref[1]) * dinv + relu1_ref[...] * (dinv * dinv)
        h2p = jnp.maximum(
            jnp.dot(agg, w2_ref[...], preferred_element_type=jnp.float32)
            + b2_ref[...], 0.0)
        bc = batch_col_ref[...]

        @pl.when(k == 0)
        def _():
            xmax_acc[...] = jnp.full((_B, H), -jnp.inf, jnp.float32)
            sum_acc[...] = jnp.zeros((_B, H), jnp.float32)
            cnt_acc[...] = jnp.zeros((_B, 1), jnp.float32)

        iota = lax.broadcasted_iota(jnp.int32, (_B, bs), 0)
        onehot = (batch_row_ref[0] == iota).astype(jnp.float32)
        sum_acc[...] += jnp.dot(onehot, h2p,
                                preferred_element_type=jnp.float32)
        cnt_acc[...] += jnp.sum(onehot, axis=1, keepdims=True)

        for b in range(_B):
            m = jnp.max(jnp.where(bc == b, h2p, -jnp.inf), axis=0,
                        keepdims=True)
            xmax_acc[b:b + 1, :] = jnp.maximum(xmax_acc[b:b + 1, :], m)

        @pl.when(k == nblocks - 1)
        def _():
            mean = sum_acc[...] / jnp.maximum(cnt_acc[...], 1.0)
            g = jnp.concatenate([mean, xmax_acc[...]], axis=1)
            gf = jnp.maximum(
                jnp.dot(g, wf1_ref[...], preferred_element_type=jnp.float32)
                + bf1_ref[...], 0.0)
            z = jnp.dot(gf, wf2_ref[...],
                        preferred_element_type=jnp.float32) + bf2_ref[...]
            out_ref[...] = 1.0 / (1.0 + jnp.exp(-z))

    return post_body


def kernel(x, edge_index, batch, W1, b1, W2, b2, Wf1, bf1, Wf2, bf2):
    N, F_in = x.shape
    H = W1.shape[1]
    E = edge_index.shape[1]

    ones_d = jnp.ones((_KD,), jnp.float32)
    zeros1 = jnp.zeros((N,), jnp.float32)
    zeros_nh = jnp.zeros((N, H), jnp.float32)

    h1 = pl.pallas_call(
        _pre_body,
        out_shape=jax.ShapeDtypeStruct((N, H), jnp.float32),
    )(x, W1)

    p1, dinv, _ = _make_layer1_kernel(N, E, H)(
        h1, edge_index, ones_d, zeros1, zeros_nh)
    dinv2 = dinv.reshape(N, 1)

    p2, relu1, _ = _make_layer2_kernel(N, E, H)(
        p1, h1, dinv, b1, edge_index, zeros_nh)

    nblocks = 10
    bs = N // nblocks
    out = pl.pallas_call(
        _make_post_body(nblocks, H),
        grid=(nblocks,),
        in_specs=[
            pl.BlockSpec((2, bs, H), lambda k: (0, k, 0)),
            pl.BlockSpec((bs, H), lambda k: (k, 0)),
            pl.BlockSpec((bs, 1), lambda k: (k, 0)),
            pl.BlockSpec((H, H), lambda k: (0, 0)),
            pl.BlockSpec((1, H), lambda k: (0, 0)),
            pl.BlockSpec((bs, 1), lambda k: (k, 0)),
            pl.BlockSpec((1, 1, bs), lambda k: (k, 0, 0)),
            pl.BlockSpec((2 * H, H), lambda k: (0, 0)),
            pl.BlockSpec((1, H), lambda k: (0, 0)),
            pl.BlockSpec((H, 1), lambda k: (0, 0)),
            pl.BlockSpec((1, 1), lambda k: (0, 0)),
        ],
        out_specs=pl.BlockSpec((_B, 1), lambda k: (0, 0)),
        out_shape=jax.ShapeDtypeStruct((_B, 1), jnp.float32),
        scratch_shapes=[
            pltpu.VMEM((_B, H), jnp.float32),
            pltpu.VMEM((_B, H), jnp.float32),
            pltpu.VMEM((_B, 1), jnp.float32),
        ],
    )(p2, relu1, dinv2, W2, b2.reshape(1, H),
      batch.reshape(N, 1), batch.reshape(nblocks, 1, bs), Wf1,
      bf1.reshape(1, H), Wf2, bf2.reshape(1, 1))

    return out


# speedup vs baseline: 1.0223x; 1.0223x over previous
import functools

import jax
import jax.numpy as jnp
from jax import lax
from jax.experimental import pallas as pl
from jax.experimental.pallas import tpu as pltpu
from jax.experimental.pallas import tpu_sc as plsc

NC = 2
NS = 16
NW = NC * NS

_B = 64

_RT = 640
_SZ = 128
_KD = 1000
_KA = 400
_NIB = 4
_NRB = 3


def _sc_mesh():
    return plsc.VectorSubcoreMesh(
        core_axis_name="c", subcore_axis_name="s", num_cores=NC,
        num_subcores=NS)


def _vrsqrt(x):
    i = lax.bitcast_convert_type(x, jnp.int32)
    i = 0x5F3759DF - lax.shift_right_logical(i, 1)
    y = lax.bitcast_convert_type(i, jnp.float32)
    for _ in range(3):
        y = y * (1.5 - 0.5 * x * y * y)
    return y


def _idx_start(ei_hbm, src_v, dst_v, isems, base, K, i):
    off = pl.multiple_of(base + i * K, K)
    j = i % _NIB
    return (
        pltpu.async_copy(ei_hbm.at[0, pl.ds(off, K)], src_v.at[j],
                         isems[j]),
        pltpu.async_copy(ei_hbm.at[1, pl.ds(off, K)], dst_v.at[j],
                         isems[j]),
    )


def _edge_pipeline(ei_hbm, table, acc_sh, src_v, dst_v, rows_v,
                   isems, gsems, ssems, base, nchunk, K, idx_d=None):
    def idx_start(i):
        return _idx_start(ei_hbm, src_v, dst_v, isems, base, K, i)

    def gather_start(i):
        return pltpu.async_copy(table.at[src_v.at[i % _NIB]],
                                rows_v.at[i % _NRB], gsems[i % _NRB])

    def scat_start(i):
        return pltpu.async_copy(rows_v.at[i % _NRB],
                                acc_sh.at[dst_v.at[i % _NIB]],
                                ssems[i % _NRB], add=True)

    if idx_d is None:
        idx_d = {i: idx_start(i) for i in range(min(3, nchunk))}
    plsc.subcore_barrier()
    gat_d = {}
    for i in range(min(2, nchunk)):
        for d in idx_d[i]:
            d.wait()
        gat_d[i] = gather_start(i)

    scat_d = {}
    for i in range(nchunk):
        gat_d[i].wait()
        if i >= 1:
            scat_d[i - 1].wait()
        scat_d[i] = scat_start(i)
        if i + 2 < nchunk:
            for d in idx_d[i + 2]:
                d.wait()
            gat_d[i + 2] = gather_start(i + 2)
        if i + 3 < nchunk:
            idx_d[i + 3] = idx_start(i + 3)
    scat_d[nchunk - 1].wait()


def _make_layer1_kernel(N, E, H):
    ept_deg = E // NS
    nch_deg = ept_deg // _KD
    ept = E // NW
    nch = ept // _KA
    nsub = _RT // _SZ

    @functools.partial(
        pl.kernel,
        out_type=(
            jax.ShapeDtypeStruct((NC, N, H), jnp.float32),
            jax.ShapeDtypeStruct((N,), jnp.float32),
            jax.ShapeDtypeStruct((NC, N, H), jnp.float32),
        ),
        mesh=_sc_mesh(),
        scratch_types=[
            pltpu.VMEM((_NIB, _KD), jnp.int32),
            pltpu.VMEM((_KD,), jnp.float32),
            pltpu.VMEM((_NIB, _KA), jnp.int32),
            pltpu.VMEM((_NIB, _KA), jnp.int32),
            pltpu.VMEM((_NRB, _KA, H), jnp.float32),
            pltpu.VMEM((_RT,), jnp.float32),
            pltpu.VMEM((_RT,), jnp.float32),
            pltpu.VMEM_SHARED((N,), jnp.float32),
            pltpu.VMEM_SHARED((N, H), jnp.float32),
        ]
        + [pltpu.SemaphoreType.DMA] * (_NIB + 2 * _NRB + 2),
        compiler_params=pltpu.CompilerParams(use_tc_tiling_on_sc=False),
    )
    def layer1(h1_hbm, ei_hbm, ones_hbm, zeros1_hbm, zerosh_hbm,
               p_hbm, dinv_hbm, tbl_hbm,
               didx_v, ones_v, src_v, dst_v, rows_v, deg_v, dinv_v,
               deg_sh, acc_sh, *sems):
        isems = sems[:_NIB]
        gsems = sems[_NIB:_NIB + _NRB]
        ssems = sems[_NIB + _NRB:_NIB + 2 * _NRB]
        zsem = sems[_NIB + 2 * _NRB]
        z2sem = sems[_NIB + 2 * _NRB + 1]
        c = lax.axis_index("c")
        s = lax.axis_index("s")
        start = pl.multiple_of(jnp.minimum(s * _RT, N - _RT), 80)

        dbase = s * ept_deg

        def didx_start(i):
            return pltpu.async_copy(
                ei_hbm.at[1, pl.ds(pl.multiple_of(dbase + i * _KD, _KD),
                                   _KD)],
                didx_v.at[i % _NIB], isems[i % _NIB])

        @pl.when(s == 0)
        def _():
            pltpu.async_copy(zeros1_hbm, deg_sh, zsem)

        didx_d = {i: didx_start(i) for i in range(min(3, nch_deg))}
        pltpu.sync_copy(ones_hbm, ones_v)
        zd = pltpu.async_copy(zerosh_hbm.at[pl.ds(start, _RT)],
                              acc_sh.at[pl.ds(start, _RT)], z2sem)

        @pl.when(s == 0)
        def _():
            pltpu.make_async_copy(zeros1_hbm, deg_sh, zsem).wait()
        plsc.subcore_barrier()

        dscat_d = {}
        for i in range(nch_deg):
            didx_d[i].wait()
            if i >= 1:
                dscat_d[i - 1].wait()
            dscat_d[i] = pltpu.async_copy(
                ones_v, deg_sh.at[didx_v.at[i % _NIB]], ssems[i % 2],
                add=True)
            if i + 3 < nch_deg:
                didx_d[i + 3] = didx_start(i + 3)
        dscat_d[nch_deg - 1].wait()

        base = (c * NS + s) * ept
        aidx_d = {i: _idx_start(ei_hbm, src_v, dst_v, isems, base, _KA, i)
                  for i in range(min(3, nch))}
        plsc.subcore_barrier()

        pltpu.sync_copy(deg_sh.at[pl.ds(start, _RT)], deg_v)
        for i in range(_RT // 16):
            sl = pl.ds(i * 16, 16)
            dinv_v[sl] = _vrsqrt(deg_v[sl] + 1.0)

        @pl.when(c == 0)
        def _():
            pltpu.sync_copy(dinv_v, dinv_hbm.at[pl.ds(start, _RT)])

        def ld_start(p):
            r0 = start + p * _SZ
            return pltpu.async_copy(
                h1_hbm.at[pl.ds(r0, _SZ)],
                rows_v.at[p % _NRB, pl.ds(0, _SZ)], gsems[p % _NRB])

        def st_start(p):
            r0 = start + p * _SZ
            return pltpu.async_copy(
                rows_v.at[p % _NRB, pl.ds(0, _SZ)],
                tbl_hbm.at[c, pl.ds(r0, _SZ)], ssems[p % _NRB])

        ld_d = {p: ld_start(p) for p in range(min(2, nsub))}
        st_d = {}
        for p in range(nsub):
            ld_d[p].wait()
            if p >= 1:
                st_d[p - 1].wait()
            if p + 2 < nsub:
                ld_d[p + 2] = ld_start(p + 2)

            def scale_grp(i, carry):
                dvec = dinv_v[pl.ds(
                    pl.multiple_of(p * _SZ + i * 16, 16), 16)]
                for j in range(16):
                    r = i * 16 + j
                    dv = dvec[j]
                    for cc in range(H // 16):
                        sl = pl.ds(cc * 16, 16)
                        rows_v[p % _NRB, r, sl] = rows_v[p % _NRB, r, sl] * dv
                return carry
            lax.fori_loop(0, _SZ // 16, scale_grp, 0)
            st_d[p] = st_start(p)
        st_d[nsub - 1].wait()
        zd.wait()

        _edge_pipeline(ei_hbm, tbl_hbm.at[c], acc_sh, src_v, dst_v, rows_v,
                       isems, gsems, ssems, base, nch, _KA, idx_d=aidx_d)

        plsc.subcore_barrier()
        pltpu.sync_copy(acc_sh.at[pl.ds(start, _RT)],
                        p_hbm.at[c, pl.ds(start, _RT)])

    return layer1


def _make_layer2_kernel(N, E, H):
    ept = E // NW
    nch = ept // _KA

    @functools.partial(
        pl.kernel,
        out_type=jax.ShapeDtypeStruct((NC, N, H), jnp.float32),
        mesh=_sc_mesh(),
        scratch_types=[
            pltpu.VMEM((_NIB, _KA), jnp.int32),
            pltpu.VMEM((_NIB, _KA), jnp.int32),
            pltpu.VMEM((_NRB, _KA, H), jnp.float32),
            pltpu.VMEM_SHARED((N, H), jnp.float32),
        ]
        + [pltpu.SemaphoreType.DMA] * (_NIB + 2 * _NRB + 1),
        compiler_params=pltpu.CompilerParams(use_tc_tiling_on_sc=False),
    )
    def layer2(tbl_hbm, ei_hbm, zerosh_hbm, p_hbm,
               src_v, dst_v, rows_v, acc_sh, *sems):
        isems = sems[:_NIB]
        gsems = sems[_NIB:_NIB + _NRB]
        ssems = sems[_NIB + _NRB:_NIB + 2 * _NRB]
        zsem = sems[_NIB + 2 * _NRB]
        c = lax.axis_index("c")
        s = lax.axis_index("s")
        start = pl.multiple_of(jnp.minimum(s * _RT, N - _RT), 80)

        zd = pltpu.async_copy(zerosh_hbm.at[pl.ds(start, _RT)],
                              acc_sh.at[pl.ds(start, _RT)], zsem)
        zd.wait()

        base = (c * NS + s) * ept
        _edge_pipeline(ei_hbm, tbl_hbm, acc_sh, src_v, dst_v, rows_v,
                       isems, gsems, ssems, base, nch, _KA)

        plsc.subcore_barrier()
        pltpu.sync_copy(acc_sh.at[pl.ds(start, _RT)],
                        p_hbm.at[c, pl.ds(start, _RT)])

    return layer2


def _pre_body(x_ref, w1_ref, h_ref):
    h_ref[...] = jnp.dot(x_ref[...], w1_ref[...],
                         preferred_element_type=jnp.float32)


def _mid_body(p_ref, h_ref, dinv_ref, b1_ref, relu1_ref, tbl_ref):
    dinv = dinv_ref[...]
    agg = (p_ref[0] + p_ref[1]) * dinv + h_ref[...] * (dinv * dinv)
    relu1 = jnp.maximum(agg + b1_ref[...], 0.0)
    relu1_ref[...] = relu1
    tbl_ref[...] = relu1 * dinv


def _make_post_body(nblocks, H):
    def post_body(p_ref, relu1_ref, dinv_ref, w2_ref, b2_ref,
                  batch_col_ref, batch_row_ref, wf1_ref, bf1_ref,
                  wf2_ref, bf2_ref, out_ref, xmax_acc, sum_acc, cnt_acc):
        k = pl.program_id(0)
        bs = relu1_ref.shape[0]

        dinv = dinv_ref[...]
        agg = (p_ref[0] + p_ref[1]) * dinv + relu1_ref[...] * (dinv * dinv)
        h2p = jnp.maximum(
            jnp.dot(agg, w2_ref[...], preferred_element_type=jnp.float32)
            + b2_ref[...], 0.0)
        bc = batch_col_ref[...]

        @pl.when(k == 0)
        def _():
            xmax_acc[...] = jnp.full((_B, H), -jnp.inf, jnp.float32)
            sum_acc[...] = jnp.zeros((_B, H), jnp.float32)
            cnt_acc[...] = jnp.zeros((_B, 1), jnp.float32)

        iota = lax.broadcasted_iota(jnp.int32, (_B, bs), 0)
        onehot = (batch_row_ref[0] == iota).astype(jnp.float32)
        sum_acc[...] += jnp.dot(onehot, h2p,
                                preferred_element_type=jnp.float32)
        cnt_acc[...] += jnp.sum(onehot, axis=1, keepdims=True)

        for b in range(_B):
            m = jnp.max(jnp.where(bc == b, h2p, -jnp.inf), axis=0,
                        keepdims=True)
            xmax_acc[b:b + 1, :] = jnp.maximum(xmax_acc[b:b + 1, :], m)

        @pl.when(k == nblocks - 1)
        def _():
            mean = sum_acc[...] / jnp.maximum(cnt_acc[...], 1.0)
            g = jnp.concatenate([mean, xmax_acc[...]], axis=1)
            gf = jnp.maximum(
                jnp.dot(g, wf1_ref[...], preferred_element_type=jnp.float32)
                + bf1_ref[...], 0.0)
            z = jnp.dot(gf, wf2_ref[...],
                        preferred_element_type=jnp.float32) + bf2_ref[...]
            out_ref[...] = 1.0 / (1.0 + jnp.exp(-z))

    return post_body


def kernel(x, edge_index, batch, W1, b1, W2, b2, Wf1, bf1, Wf2, bf2):
    N, F_in = x.shape
    H = W1.shape[1]
    E = edge_index.shape[1]

    ones_d = jnp.ones((_KD,), jnp.float32)
    zeros1 = jnp.zeros((N,), jnp.float32)
    zeros_nh = jnp.zeros((N, H), jnp.float32)

    h1 = pl.pallas_call(
        _pre_body,
        out_shape=jax.ShapeDtypeStruct((N, H), jnp.float32),
    )(x, W1)

    p1, dinv, _ = _make_layer1_kernel(N, E, H)(
        h1, edge_index, ones_d, zeros1, zeros_nh)
    dinv2 = dinv.reshape(N, 1)

    relu1, tbl2 = pl.pallas_call(
        _mid_body,
        out_shape=(
            jax.ShapeDtypeStruct((N, H), jnp.float32),
            jax.ShapeDtypeStruct((N, H), jnp.float32),
        ),
    )(p1, h1, dinv2, b1.reshape(1, H))

    p2 = _make_layer2_kernel(N, E, H)(tbl2, edge_index, zeros_nh)

    nblocks = 10
    bs = N // nblocks
    out = pl.pallas_call(
        _make_post_body(nblocks, H),
        grid=(nblocks,),
        in_specs=[
            pl.BlockSpec((2, bs, H), lambda k: (0, k, 0)),
            pl.BlockSpec((bs, H), lambda k: (k, 0)),
            pl.BlockSpec((bs, 1), lambda k: (k, 0)),
            pl.BlockSpec((H, H), lambda k: (0, 0)),
            pl.BlockSpec((1, H), lambda k: (0, 0)),
            pl.BlockSpec((bs, 1), lambda k: (k, 0)),
            pl.BlockSpec((1, 1, bs), lambda k: (k, 0, 0)),
            pl.BlockSpec((2 * H, H), lambda k: (0, 0)),
            pl.BlockSpec((1, H), lambda k: (0, 0)),
            pl.BlockSpec((H, 1), lambda k: (0, 0)),
            pl.BlockSpec((1, 1), lambda k: (0, 0)),
        ],
        out_specs=pl.BlockSpec((_B, 1), lambda k: (0, 0)),
        out_shape=jax.ShapeDtypeStruct((_B, 1), jnp.float32),
        scratch_shapes=[
            pltpu.VMEM((_B, H), jnp.float32),
            pltpu.VMEM((_B, H), jnp.float32),
            pltpu.VMEM((_B, 1), jnp.float32),
        ],
    )(p2, relu1, dinv2, W2, b2.reshape(1, H),
      batch.reshape(N, 1), batch.reshape(nblocks, 1, bs), Wf1,
      bf1.reshape(1, H), Wf2, bf2.reshape(1, 1))

    return out


# speedup vs baseline: 1.0257x; 1.0033x over previous
import functools

import jax
import jax.numpy as jnp
from jax import lax
from jax.experimental import pallas as pl
from jax.experimental.pallas import tpu as pltpu
from jax.experimental.pallas import tpu_sc as plsc

NC = 2
NS = 16
NW = NC * NS

_B = 64

_RT = 640
_SZ = 128
_KD = 1000
_KA = 400
_NIB = 4
_NRB = 3


def _sc_mesh():
    return plsc.VectorSubcoreMesh(
        core_axis_name="c", subcore_axis_name="s", num_cores=NC,
        num_subcores=NS)


def _vrsqrt(x):
    i = lax.bitcast_convert_type(x, jnp.int32)
    i = 0x5F3759DF - lax.shift_right_logical(i, 1)
    y = lax.bitcast_convert_type(i, jnp.float32)
    for _ in range(3):
        y = y * (1.5 - 0.5 * x * y * y)
    return y


def _idx_start(ei_hbm, src_v, dst_v, isems, base, K, i):
    off = pl.multiple_of(base + i * K, K)
    j = i % _NIB
    return (
        pltpu.async_copy(ei_hbm.at[0, pl.ds(off, K)], src_v.at[j],
                         isems[j]),
        pltpu.async_copy(ei_hbm.at[1, pl.ds(off, K)], dst_v.at[j],
                         isems[j]),
    )


def _edge_pipeline(ei_hbm, table, acc_sh, src_v, dst_v, rows_v,
                   isems, gsems, ssems, base, nchunk, K, idx_d=None):
    def idx_start(i):
        return _idx_start(ei_hbm, src_v, dst_v, isems, base, K, i)

    def gather_start(i):
        return pltpu.async_copy(table.at[src_v.at[i % _NIB]],
                                rows_v.at[i % _NRB], gsems[i % _NRB])

    def scat_start(i):
        return pltpu.async_copy(rows_v.at[i % _NRB],
                                acc_sh.at[dst_v.at[i % _NIB]],
                                ssems[i % _NRB], add=True)

    if idx_d is None:
        idx_d = {i: idx_start(i) for i in range(min(3, nchunk))}
    plsc.subcore_barrier()
    gat_d = {}
    for i in range(min(2, nchunk)):
        for d in idx_d[i]:
            d.wait()
        gat_d[i] = gather_start(i)

    scat_d = {}
    for i in range(nchunk):
        gat_d[i].wait()
        if i >= 1:
            scat_d[i - 1].wait()
        scat_d[i] = scat_start(i)
        if i + 2 < nchunk:
            for d in idx_d[i + 2]:
                d.wait()
            gat_d[i + 2] = gather_start(i + 2)
        if i + 3 < nchunk:
            idx_d[i + 3] = idx_start(i + 3)
    scat_d[nchunk - 1].wait()


def _make_layer1_kernel(N, E, H):
    ept_deg = E // NS
    nch_deg = ept_deg // _KD
    ept = E // NW
    nch = ept // _KA
    nsub = _RT // _SZ

    @functools.partial(
        pl.kernel,
        out_type=(
            jax.ShapeDtypeStruct((NC, N, H), jnp.float32),
            jax.ShapeDtypeStruct((N,), jnp.float32),
            jax.ShapeDtypeStruct((NC, N, H), jnp.float32),
        ),
        mesh=_sc_mesh(),
        scratch_types=[
            pltpu.VMEM((_NIB, _KD), jnp.int32),
            pltpu.VMEM((_KD,), jnp.float32),
            pltpu.VMEM((_NIB, _KA), jnp.int32),
            pltpu.VMEM((_NIB, _KA), jnp.int32),
            pltpu.VMEM((_NRB, _KA, H), jnp.float32),
            pltpu.VMEM((_RT,), jnp.float32),
            pltpu.VMEM((_RT,), jnp.float32),
            pltpu.VMEM_SHARED((N,), jnp.float32),
            pltpu.VMEM_SHARED((N, H), jnp.float32),
        ]
        + [pltpu.SemaphoreType.DMA] * (_NIB + 2 * _NRB + 2),
        compiler_params=pltpu.CompilerParams(use_tc_tiling_on_sc=False),
    )
    def layer1(h1_hbm, ei_hbm, ones_hbm, zeros1_hbm, zerosh_hbm,
               p_hbm, dinv_hbm, tbl_hbm,
               didx_v, ones_v, src_v, dst_v, rows_v, deg_v, dinv_v,
               deg_sh, acc_sh, *sems):
        isems = sems[:_NIB]
        gsems = sems[_NIB:_NIB + _NRB]
        ssems = sems[_NIB + _NRB:_NIB + 2 * _NRB]
        zsem = sems[_NIB + 2 * _NRB]
        z2sem = sems[_NIB + 2 * _NRB + 1]
        c = lax.axis_index("c")
        s = lax.axis_index("s")
        start = pl.multiple_of(jnp.minimum(s * _RT, N - _RT), 80)

        dbase = s * ept_deg

        def didx_start(i):
            return pltpu.async_copy(
                ei_hbm.at[1, pl.ds(pl.multiple_of(dbase + i * _KD, _KD),
                                   _KD)],
                didx_v.at[i % _NIB], isems[i % _NIB])

        @pl.when(s == 0)
        def _():
            pltpu.async_copy(zeros1_hbm, deg_sh, zsem)

        def ld_start(p):
            r0 = start + p * _SZ
            return pltpu.async_copy(
                h1_hbm.at[pl.ds(r0, _SZ)],
                rows_v.at[p % _NRB, pl.ds(0, _SZ)], gsems[p % _NRB])

        didx_d = {i: didx_start(i) for i in range(min(3, nch_deg))}
        ld_d = {p: ld_start(p) for p in range(min(2, nsub))}
        pltpu.sync_copy(ones_hbm, ones_v)
        zd = pltpu.async_copy(zerosh_hbm.at[pl.ds(start, _RT)],
                              acc_sh.at[pl.ds(start, _RT)], z2sem)

        @pl.when(s == 0)
        def _():
            pltpu.make_async_copy(zeros1_hbm, deg_sh, zsem).wait()
        plsc.subcore_barrier()

        dscat_d = {}
        for i in range(nch_deg):
            didx_d[i].wait()
            if i >= 1:
                dscat_d[i - 1].wait()
            dscat_d[i] = pltpu.async_copy(
                ones_v, deg_sh.at[didx_v.at[i % _NIB]], ssems[i % 2],
                add=True)
            if i + 3 < nch_deg:
                didx_d[i + 3] = didx_start(i + 3)
        dscat_d[nch_deg - 1].wait()

        base = (c * NS + s) * ept
        aidx_d = {i: _idx_start(ei_hbm, src_v, dst_v, isems, base, _KA, i)
                  for i in range(min(3, nch))}
        plsc.subcore_barrier()

        pltpu.sync_copy(deg_sh.at[pl.ds(start, _RT)], deg_v)
        for i in range(_RT // 16):
            sl = pl.ds(i * 16, 16)
            dinv_v[sl] = _vrsqrt(deg_v[sl] + 1.0)

        @pl.when(c == 0)
        def _():
            pltpu.sync_copy(dinv_v, dinv_hbm.at[pl.ds(start, _RT)])

        def st_start(p):
            r0 = start + p * _SZ
            return pltpu.async_copy(
                rows_v.at[p % _NRB, pl.ds(0, _SZ)],
                tbl_hbm.at[c, pl.ds(r0, _SZ)], ssems[p % _NRB])

        st_d = {}
        for p in range(nsub):
            ld_d[p].wait()
            if p >= 1:
                st_d[p - 1].wait()
            if p + 2 < nsub:
                ld_d[p + 2] = ld_start(p + 2)

            def scale_grp(i, carry):
                dvec = dinv_v[pl.ds(
                    pl.multiple_of(p * _SZ + i * 16, 16), 16)]
                for j in range(16):
                    r = i * 16 + j
                    dv = dvec[j]
                    for cc in range(H // 16):
                        sl = pl.ds(cc * 16, 16)
                        rows_v[p % _NRB, r, sl] = rows_v[p % _NRB, r, sl] * dv
                return carry
            lax.fori_loop(0, _SZ // 16, scale_grp, 0)
            st_d[p] = st_start(p)
        st_d[nsub - 1].wait()
        zd.wait()

        _edge_pipeline(ei_hbm, tbl_hbm.at[c], acc_sh, src_v, dst_v, rows_v,
                       isems, gsems, ssems, base, nch, _KA, idx_d=aidx_d)

        plsc.subcore_barrier()
        pltpu.sync_copy(acc_sh.at[pl.ds(start, _RT)],
                        p_hbm.at[c, pl.ds(start, _RT)])

    return layer1


def _make_layer2_kernel(N, E, H):
    ept = E // NW
    nch = ept // _KA

    @functools.partial(
        pl.kernel,
        out_type=jax.ShapeDtypeStruct((NC, N, H), jnp.float32),
        mesh=_sc_mesh(),
        scratch_types=[
            pltpu.VMEM((_NIB, _KA), jnp.int32),
            pltpu.VMEM((_NIB, _KA), jnp.int32),
            pltpu.VMEM((_NRB, _KA, H), jnp.float32),
            pltpu.VMEM_SHARED((N, H), jnp.float32),
        ]
        + [pltpu.SemaphoreType.DMA] * (_NIB + 2 * _NRB + 1),
        compiler_params=pltpu.CompilerParams(use_tc_tiling_on_sc=False),
    )
    def layer2(tbl_hbm, ei_hbm, zerosh_hbm, p_hbm,
               src_v, dst_v, rows_v, acc_sh, *sems):
        isems = sems[:_NIB]
        gsems = sems[_NIB:_NIB + _NRB]
        ssems = sems[_NIB + _NRB:_NIB + 2 * _NRB]
        zsem = sems[_NIB + 2 * _NRB]
        c = lax.axis_index("c")
        s = lax.axis_index("s")
        start = pl.multiple_of(jnp.minimum(s * _RT, N - _RT), 80)

        zd = pltpu.async_copy(zerosh_hbm.at[pl.ds(start, _RT)],
                              acc_sh.at[pl.ds(start, _RT)], zsem)
        zd.wait()

        base = (c * NS + s) * ept
        _edge_pipeline(ei_hbm, tbl_hbm, acc_sh, src_v, dst_v, rows_v,
                       isems, gsems, ssems, base, nch, _KA)

        plsc.subcore_barrier()
        pltpu.sync_copy(acc_sh.at[pl.ds(start, _RT)],
                        p_hbm.at[c, pl.ds(start, _RT)])

    return layer2


def _pre_body(x_ref, w1_ref, h_ref):
    h_ref[...] = jnp.dot(x_ref[...], w1_ref[...],
                         preferred_element_type=jnp.float32)


def _mid_body(p_ref, h_ref, dinv_ref, b1_ref, relu1_ref, tbl_ref):
    dinv = dinv_ref[...]
    agg = (p_ref[0] + p_ref[1]) * dinv + h_ref[...] * (dinv * dinv)
    relu1 = jnp.maximum(agg + b1_ref[...], 0.0)
    relu1_ref[...] = relu1
    tbl_ref[...] = relu1 * dinv


def _make_post_body(nblocks, H):
    def post_body(p_ref, relu1_ref, dinv_ref, w2_ref, b2_ref,
                  batch_col_ref, batch_row_ref, wf1_ref, bf1_ref,
                  wf2_ref, bf2_ref, out_ref, xmax_acc, sum_acc, cnt_acc):
        k = pl.program_id(0)
        bs = relu1_ref.shape[0]

        dinv = dinv_ref[...]
        agg = (p_ref[0] + p_ref[1]) * dinv + relu1_ref[...] * (dinv * dinv)
        h2p = jnp.maximum(
            jnp.dot(agg, w2_ref[...], preferred_element_type=jnp.float32)
            + b2_ref[...], 0.0)
        bc = batch_col_ref[...]

        @pl.when(k == 0)
        def _():
            xmax_acc[...] = jnp.full((_B, H), -jnp.inf, jnp.float32)
            sum_acc[...] = jnp.zeros((_B, H), jnp.float32)
            cnt_acc[...] = jnp.zeros((_B, 1), jnp.float32)

        iota = lax.broadcasted_iota(jnp.int32, (_B, bs), 0)
        onehot = (batch_row_ref[0] == iota).astype(jnp.float32)
        sum_acc[...] += jnp.dot(onehot, h2p,
                                preferred_element_type=jnp.float32)
        cnt_acc[...] += jnp.sum(onehot, axis=1, keepdims=True)

        for b in range(_B):
            m = jnp.max(jnp.where(bc == b, h2p, -jnp.inf), axis=0,
                        keepdims=True)
            xmax_acc[b:b + 1, :] = jnp.maximum(xmax_acc[b:b + 1, :], m)

        @pl.when(k == nblocks - 1)
        def _():
            mean = sum_acc[...] / jnp.maximum(cnt_acc[...], 1.0)
            g = jnp.concatenate([mean, xmax_acc[...]], axis=1)
            gf = jnp.maximum(
                jnp.dot(g, wf1_ref[...], preferred_element_type=jnp.float32)
                + bf1_ref[...], 0.0)
            z = jnp.dot(gf, wf2_ref[...],
                        preferred_element_type=jnp.float32) + bf2_ref[...]
            out_ref[...] = 1.0 / (1.0 + jnp.exp(-z))

    return post_body


def kernel(x, edge_index, batch, W1, b1, W2, b2, Wf1, bf1, Wf2, bf2):
    N, F_in = x.shape
    H = W1.shape[1]
    E = edge_index.shape[1]

    ones_d = jnp.ones((_KD,), jnp.float32)
    zeros1 = jnp.zeros((N,), jnp.float32)
    zeros_nh = jnp.zeros((N, H), jnp.float32)

    h1 = pl.pallas_call(
        _pre_body,
        out_shape=jax.ShapeDtypeStruct((N, H), jnp.float32),
    )(x, W1)

    p1, dinv, _ = _make_layer1_kernel(N, E, H)(
        h1, edge_index, ones_d, zeros1, zeros_nh)
    dinv2 = dinv.reshape(N, 1)

    relu1, tbl2 = pl.pallas_call(
        _mid_body,
        out_shape=(
            jax.ShapeDtypeStruct((N, H), jnp.float32),
            jax.ShapeDtypeStruct((N, H), jnp.float32),
        ),
    )(p1, h1, dinv2, b1.reshape(1, H))

    p2 = _make_layer2_kernel(N, E, H)(tbl2, edge_index, zeros_nh)

    nblocks = 10
    bs = N // nblocks
    out = pl.pallas_call(
        _make_post_body(nblocks, H),
        grid=(nblocks,),
        in_specs=[
            pl.BlockSpec((2, bs, H), lambda k: (0, k, 0)),
            pl.BlockSpec((bs, H), lambda k: (k, 0)),
            pl.BlockSpec((bs, 1), lambda k: (k, 0)),
            pl.BlockSpec((H, H), lambda k: (0, 0)),
            pl.BlockSpec((1, H), lambda k: (0, 0)),
            pl.BlockSpec((bs, 1), lambda k: (k, 0)),
            pl.BlockSpec((1, 1, bs), lambda k: (k, 0, 0)),
            pl.BlockSpec((2 * H, H), lambda k: (0, 0)),
            pl.BlockSpec((1, H), lambda k: (0, 0)),
            pl.BlockSpec((H, 1), lambda k: (0, 0)),
            pl.BlockSpec((1, 1), lambda k: (0, 0)),
        ],
        out_specs=pl.BlockSpec((_B, 1), lambda k: (0, 0)),
        out_shape=jax.ShapeDtypeStruct((_B, 1), jnp.float32),
        scratch_shapes=[
            pltpu.VMEM((_B, H), jnp.float32),
            pltpu.VMEM((_B, H), jnp.float32),
            pltpu.VMEM((_B, 1), jnp.float32),
        ],
    )(p2, relu1, dinv2, W2, b2.reshape(1, H),
      batch.reshape(N, 1), batch.reshape(nblocks, 1, bs), Wf1,
      bf1.reshape(1, H), Wf2, bf2.reshape(1, 1))

    return out
